# baseline (speedup 1.0000x reference)
import functools

import jax
import jax.numpy as jnp
from jax.experimental import pallas as pl
from jax.experimental.pallas import tpu as pltpu

_B = 128
_D = 5
_N = 31
_OP = 16
_PRED = 512
_FEAT = 64
_HID = 128
_BITMAP = 1000
_REP = 128

_NDATA = 5


def _dotb(a, b):
    return jax.lax.dot_general(
        a.astype(jnp.bfloat16), b.astype(jnp.bfloat16),
        (((1,), (0,)), ((), ())), preferred_element_type=jnp.float32
    )


def _dot32(a, b):
    return jax.lax.dot_general(
        a, b, (((1,), (0,)), ((), ())), preferred_element_type=jnp.float32
    )


def _row(b_ref):
    return b_ref[...].reshape(1, -1)


def _tree_body(
    op_hbm, feat_hbm, c1_hbm, c2_hbm, bm_hbm,
    Wop_ref, Wfeat_ref, Wp_ref, Wbm_ref, Wr1b_ref, br1_ref,
    Wch_ref, W2b_ref, W2_ref, b2_ref, W3b_ref, W3_ref, b3_ref,
    W_h21_ref, b_h21_ref, W_h31_ref, b_h31_ref, W_o1_ref, b_o1_ref,
    W_h22_ref, b_h22_ref, W_h32_ref, b_h32_ref, W_o2_ref, b_o2_ref,
    cost_ref, card_ref,
    op_buf, feat_buf, c1_buf, c2_buf, bm_buf, sems, z_sc,
):
    i = pl.program_id(0)
    slot = jax.lax.rem(i, 2)
    nslot = jax.lax.rem(i + 1, 2)
    hbms = [op_hbm, feat_hbm, c1_hbm, c2_hbm, bm_hbm]
    bufs = [op_buf, feat_buf, c1_buf, c2_buf, bm_buf]

    def start_fetch(node, s):
        for k in range(_NDATA):
            pltpu.make_async_copy(
                hbms[k].at[node], bufs[k].at[s], sems.at[s, k]
            ).start()

    @pl.when(i == 0)
    def _warmup():
        start_fetch(0, 0)

    @pl.when(i + 1 < _N)
    def _prefetch():
        start_fetch(i + 1, nslot)

    for k in range(_NDATA):
        pltpu.make_async_copy(
            hbms[k].at[i], bufs[k].at[slot], sems.at[slot, k]
        ).wait()

    Wr1b = Wr1b_ref[...]
    A0 = Wr1b[0 * _HID:1 * _HID]
    A1 = Wr1b[1 * _HID:2 * _HID]
    A2 = Wr1b[2 * _HID:3 * _HID]
    A3 = Wr1b[3 * _HID:4 * _HID]
    A4 = Wr1b[4 * _HID:5 * _HID]

    op_v = _dotb(op_buf[slot], Wop_ref[...])
    feat_v = _dotb(feat_buf[slot], Wfeat_ref[...])
    c1 = _dotb(c1_buf[slot], Wp_ref[...])
    c2 = _dotb(c2_buf[slot], Wp_ref[...])
    bmE = _dotb(bm_buf[slot], Wbm_ref[...])
    z = _dotb(op_v, A0)
    z = z + _dotb(feat_v, A1)
    z = z + _dotb(c1, A2)
    z = z + _dotb(c2, A3)
    z = z + _dotb(bmE, A4)
    z = z + _row(br1_ref)
    z_sc[pl.ds(i * _B, _B), :] = z

    @pl.when(i == _N - 1)
    def _stage2():
        cb = 5 * _HID
        Wchb = Wr1b[cb:]
        Wch32 = Wch_ref[...]

        reps = [None] * _D
        for l in range(_D - 1, -1, -1):
            bf = l >= _D - 2
            dot = _dotb if bf else _dot32
            Wc = Wchb if bf else Wch32
            W2 = W2b_ref[...] if bf else W2_ref[...]
            W3 = W3b_ref[...] if bf else W3_ref[...]
            b2 = _row(b2_ref)
            b3 = _row(b3_ref)
            n = 1 << l
            a = n - 1
            zl = z_sc[a * _B:(a + n) * _B, :]
            if l <= _D - 2:
                C = reps[l + 1].reshape(n, 2, _B, _REP)
                left = C[:, 0].reshape(n * _B, _REP)
                right = C[:, 1].reshape(n * _B, _REP)
                zl = (zl + dot(left, Wc[0 * _REP:1 * _REP])
                      + dot(right, Wc[1 * _REP:2 * _REP]))
            if l <= _D - 3:
                G = reps[l + 2].reshape(n, 4, _B, _REP)
                zl = (zl
                      + dot(G[:, 0].reshape(n * _B, _REP), Wc[2 * _REP:3 * _REP])
                      + dot(G[:, 1].reshape(n * _B, _REP), Wc[3 * _REP:4 * _REP])
                      + dot(G[:, 2].reshape(n * _B, _REP), Wc[4 * _REP:5 * _REP])
                      + dot(G[:, 3].reshape(n * _B, _REP), Wc[5 * _REP:6 * _REP]))
            h = jnp.maximum(zl, 0.0)
            h = jnp.maximum(dot(h, W2) + b2, 0.0)
            h = jnp.maximum(dot(h, W3) + b3, 0.0)
            reps[l] = h

        root = reps[0]
        cost = jnp.maximum(_dot32(root, W_h21_ref[...]) + _row(b_h21_ref), 0.0)
        cost = jnp.maximum(_dot32(cost, W_h31_ref[...]) + _row(b_h31_ref), 0.0)
        cost_ref[...] = jax.nn.sigmoid(_dot32(cost, W_o1_ref[...]) + _row(b_o1_ref))
        card = jnp.maximum(_dot32(root, W_h22_ref[...]) + _row(b_h22_ref), 0.0)
        card = jnp.maximum(_dot32(card, W_h32_ref[...]) + _row(b_h32_ref), 0.0)
        card_ref[...] = jax.nn.sigmoid(_dot32(card, W_o2_ref[...]) + _row(b_o2_ref))


@jax.jit
def kernel(op_x, feat_x, cond1_x, cond2_x, bitmap_x, has_cond,
           W_op, b_op, W_pred, b_pred, W_bm, b_bm, W_feat, b_feat,
           W_r1, b_r1, W_r2, b_r2, W_r3, b_r3,
           W_h21, b_h21, W_h31, b_h31, W_o1, b_o1,
           W_h22, b_h22, W_h32, b_h32, W_o2, b_o2):
    bf16 = jnp.bfloat16
    ones = jnp.ones((_B, _N, 1), jnp.float32)
    hc1 = has_cond[:, :, None]

    tr = lambda x: x.astype(bf16).transpose(1, 0, 2)
    opb = tr(jnp.concatenate([op_x, ones], -1))
    featb = tr(jnp.concatenate([feat_x, ones], -1))
    c1b = tr(jnp.concatenate([cond1_x, ones], -1))
    c2b = tr(jnp.concatenate([cond2_x, ones], -1))
    bmb = tr(jnp.concatenate([bitmap_x * hc1, hc1], -1))

    Wopx = jnp.concatenate([W_op, b_op[None]], 0).astype(bf16)
    Wfeatx = jnp.concatenate([W_feat, b_feat[None]], 0).astype(bf16)
    Wpx = jnp.concatenate([W_pred, b_pred[None]], 0).astype(bf16)
    Wbmx = jnp.concatenate([W_bm, b_bm[None]], 0).astype(bf16)
    Wr1b = W_r1.astype(bf16)
    Wch = W_r1[5 * _HID:]

    data = [opb, featb, c1b, c2b, bmb]
    weights = [Wopx, Wfeatx, Wpx, Wbmx, Wr1b, b_r1,
               Wch, W_r2.astype(bf16), W_r2, b_r2,
               W_r3.astype(bf16), W_r3, b_r3,
               W_h21, b_h21, W_h31, b_h31, W_o1, b_o1,
               W_h22, b_h22, W_h32, b_h32, W_o2, b_o2]

    hbm_spec = pl.BlockSpec(memory_space=pltpu.MemorySpace.HBM)

    def w_spec(shape):
        nd = len(shape)
        return pl.BlockSpec(tuple(shape), lambda i, _nd=nd: (0,) * _nd)

    in_specs = [hbm_spec] * _NDATA + [w_spec(w.shape) for w in weights]

    out_shape = (
        jax.ShapeDtypeStruct((_B, 1), jnp.float32),
        jax.ShapeDtypeStruct((_B, 1), jnp.float32),
    )
    out_specs = (
        pl.BlockSpec((_B, 1), lambda i: (0, 0)),
        pl.BlockSpec((_B, 1), lambda i: (0, 0)),
    )

    scratch_shapes = [
        pltpu.VMEM((2, _B, _OP + 1), bf16),
        pltpu.VMEM((2, _B, _FEAT + 1), bf16),
        pltpu.VMEM((2, _B, _PRED + 1), bf16),
        pltpu.VMEM((2, _B, _PRED + 1), bf16),
        pltpu.VMEM((2, _B, _BITMAP + 1), bf16),
        pltpu.SemaphoreType.DMA((2, _NDATA)),
        pltpu.VMEM((_N * _B, 512), jnp.float32),
    ]

    cost, card = pl.pallas_call(
        _tree_body,
        grid=(_N,),
        in_specs=in_specs,
        out_specs=out_specs,
        out_shape=out_shape,
        scratch_shapes=scratch_shapes,
        compiler_params=pltpu.CompilerParams(
            dimension_semantics=("arbitrary",),
        ),
    )(*data, *weights)
    return (cost, card)


# speedup vs baseline: 1.3590x; 1.3590x over previous
import functools

import jax
import jax.numpy as jnp
from jax.experimental import pallas as pl
from jax.experimental.pallas import tpu as pltpu

_B = 128
_D = 5
_N = 31
_OP = 16
_PRED = 512
_FEAT = 64
_HID = 128
_BITMAP = 1000
_REP = 128

_NDATA = 6
_FPAD = (128, 128, _PRED, _PRED, 1024, _HID)


def _dotb(a, b):
    return jax.lax.dot_general(
        a.astype(jnp.bfloat16), b.astype(jnp.bfloat16),
        (((1,), (0,)), ((), ())), preferred_element_type=jnp.float32
    )


def _dot32(a, b):
    return jax.lax.dot_general(
        a, b, (((1,), (0,)), ((), ())), preferred_element_type=jnp.float32
    )


def _row(b_ref):
    return b_ref[...].reshape(1, -1)


def _tree_body(
    op_hbm, feat_hbm, c1_hbm, c2_hbm, bm_hbm, hc_hbm,
    Wop_ref, bop_ref, Wfeat_ref, bfeat_ref, Wp_ref, bp_ref,
    Wbm_ref, bbm_ref, Wr1b_ref, br1_ref,
    Wch_ref, W2b_ref, W2_ref, b2_ref, W3b_ref, W3_ref, b3_ref,
    W_h21_ref, b_h21_ref, W_h31_ref, b_h31_ref, W_o1_ref, b_o1_ref,
    W_h22_ref, b_h22_ref, W_h32_ref, b_h32_ref, W_o2_ref, b_o2_ref,
    cost_ref, card_ref,
    op_buf, feat_buf, c1_buf, c2_buf, bm_buf, hc_buf, sems, z_sc,
):
    i = pl.program_id(0)
    slot = jax.lax.rem(i, 2)
    nslot = jax.lax.rem(i + 1, 2)
    hbms = [op_hbm, feat_hbm, c1_hbm, c2_hbm, bm_hbm, hc_hbm]
    bufs = [op_buf, feat_buf, c1_buf, c2_buf, bm_buf, hc_buf]

    def start_fetch(node, s):
        for k in range(_NDATA):
            fp = _FPAD[k]
            pltpu.make_async_copy(
                hbms[k].at[:, pl.ds(node * fp, fp)], bufs[k].at[s],
                sems.at[s, k]
            ).start()

    @pl.when(i == 0)
    def _warmup():
        start_fetch(0, 0)

    @pl.when(i + 1 < _N)
    def _prefetch():
        start_fetch(i + 1, nslot)

    for k in range(_NDATA):
        fp = _FPAD[k]
        pltpu.make_async_copy(
            hbms[k].at[:, pl.ds(i * fp, fp)], bufs[k].at[slot],
            sems.at[slot, k]
        ).wait()

    Wr1b = Wr1b_ref[...]
    A0 = Wr1b[0 * _HID:1 * _HID]
    A1 = Wr1b[1 * _HID:2 * _HID]
    A2 = Wr1b[2 * _HID:3 * _HID]
    A3 = Wr1b[3 * _HID:4 * _HID]
    A4 = Wr1b[4 * _HID:5 * _HID]

    op_v = _dotb(op_buf[slot], Wop_ref[...]) + _row(bop_ref)
    feat_v = _dotb(feat_buf[slot], Wfeat_ref[...]) + _row(bfeat_ref)
    bp = _row(bp_ref)
    c1 = _dotb(c1_buf[slot], Wp_ref[...]) + bp
    c2 = _dotb(c2_buf[slot], Wp_ref[...]) + bp
    bmE = ((_dotb(bm_buf[slot], Wbm_ref[...]) + _row(bbm_ref))
           * hc_buf[slot].astype(jnp.float32))
    z = _dotb(op_v, A0)
    z = z + _dotb(feat_v, A1)
    z = z + _dotb(c1, A2)
    z = z + _dotb(c2, A3)
    z = z + _dotb(bmE, A4)
    z = z + _row(br1_ref)
    z_sc[pl.ds(i * _B, _B), :] = z

    @pl.when(i == _N - 1)
    def _stage2():
        cb = 5 * _HID
        Wchb = Wr1b[cb:]
        Wch32 = Wch_ref[...]

        reps = [None] * _D
        for l in range(_D - 1, -1, -1):
            bf = l >= _D - 2
            dot = _dotb if bf else _dot32
            Wc = Wchb if bf else Wch32
            W2 = W2b_ref[...] if bf else W2_ref[...]
            W3 = W3b_ref[...] if bf else W3_ref[...]
            b2 = _row(b2_ref)
            b3 = _row(b3_ref)
            n = 1 << l
            a = n - 1
            zl = z_sc[a * _B:(a + n) * _B, :]
            if l <= _D - 2:
                C = reps[l + 1].reshape(n, 2, _B, _REP)
                left = C[:, 0].reshape(n * _B, _REP)
                right = C[:, 1].reshape(n * _B, _REP)
                zl = (zl + dot(left, Wc[0 * _REP:1 * _REP])
                      + dot(right, Wc[1 * _REP:2 * _REP]))
            if l <= _D - 3:
                G = reps[l + 2].reshape(n, 4, _B, _REP)
                zl = (zl
                      + dot(G[:, 0].reshape(n * _B, _REP), Wc[2 * _REP:3 * _REP])
                      + dot(G[:, 1].reshape(n * _B, _REP), Wc[3 * _REP:4 * _REP])
                      + dot(G[:, 2].reshape(n * _B, _REP), Wc[4 * _REP:5 * _REP])
                      + dot(G[:, 3].reshape(n * _B, _REP), Wc[5 * _REP:6 * _REP]))
            h = jnp.maximum(zl, 0.0)
            h = jnp.maximum(dot(h, W2) + b2, 0.0)
            h = jnp.maximum(dot(h, W3) + b3, 0.0)
            reps[l] = h

        root = reps[0]
        cost = jnp.maximum(_dot32(root, W_h21_ref[...]) + _row(b_h21_ref), 0.0)
        cost = jnp.maximum(_dot32(cost, W_h31_ref[...]) + _row(b_h31_ref), 0.0)
        cost_ref[...] = jax.nn.sigmoid(_dot32(cost, W_o1_ref[...]) + _row(b_o1_ref))
        card = jnp.maximum(_dot32(root, W_h22_ref[...]) + _row(b_h22_ref), 0.0)
        card = jnp.maximum(_dot32(card, W_h32_ref[...]) + _row(b_h32_ref), 0.0)
        card_ref[...] = jax.nn.sigmoid(_dot32(card, W_o2_ref[...]) + _row(b_o2_ref))


@jax.jit
def kernel(op_x, feat_x, cond1_x, cond2_x, bitmap_x, has_cond,
           W_op, b_op, W_pred, b_pred, W_bm, b_bm, W_feat, b_feat,
           W_r1, b_r1, W_r2, b_r2, W_r3, b_r3,
           W_h21, b_h21, W_h31, b_h31, W_o1, b_o1,
           W_h22, b_h22, W_h32, b_h32, W_o2, b_o2):
    bf16 = jnp.bfloat16

    def flat(x, fp):
        f = x.shape[2]
        if f < fp:
            x = jnp.pad(x, ((0, 0), (0, 0), (0, fp - f)))
        return x.reshape(_B, _N * fp).astype(bf16)

    opb = flat(op_x, 128)
    featb = flat(feat_x, 128)
    c1b = flat(cond1_x, _PRED)
    c2b = flat(cond2_x, _PRED)
    bmb = flat(bitmap_x, 1024)
    hcb = flat(jnp.broadcast_to(has_cond[:, :, None], (_B, _N, _HID)), _HID)

    Wopx = jnp.pad(W_op, ((0, 128 - _OP), (0, 0))).astype(bf16)
    Wfeatx = jnp.pad(W_feat, ((0, 128 - _FEAT), (0, 0))).astype(bf16)
    Wpx = W_pred.astype(bf16)
    Wbmx = jnp.pad(W_bm, ((0, 1024 - _BITMAP), (0, 0))).astype(bf16)
    Wr1b = W_r1.astype(bf16)
    Wch = W_r1[5 * _HID:]

    data = [opb, featb, c1b, c2b, bmb, hcb]
    weights = [Wopx, b_op, Wfeatx, b_feat, Wpx, b_pred, Wbmx, b_bm,
               Wr1b, b_r1,
               Wch, W_r2.astype(bf16), W_r2, b_r2,
               W_r3.astype(bf16), W_r3, b_r3,
               W_h21, b_h21, W_h31, b_h31, W_o1, b_o1,
               W_h22, b_h22, W_h32, b_h32, W_o2, b_o2]

    hbm_spec = pl.BlockSpec(memory_space=pltpu.MemorySpace.HBM)

    def w_spec(shape):
        nd = len(shape)
        return pl.BlockSpec(tuple(shape), lambda i, _nd=nd: (0,) * _nd)

    in_specs = [hbm_spec] * _NDATA + [w_spec(w.shape) for w in weights]

    out_shape = (
        jax.ShapeDtypeStruct((_B, 1), jnp.float32),
        jax.ShapeDtypeStruct((_B, 1), jnp.float32),
    )
    out_specs = (
        pl.BlockSpec((_B, 1), lambda i: (0, 0)),
        pl.BlockSpec((_B, 1), lambda i: (0, 0)),
    )

    scratch_shapes = [
        pltpu.VMEM((2, _B, _FPAD[0]), bf16),
        pltpu.VMEM((2, _B, _FPAD[1]), bf16),
        pltpu.VMEM((2, _B, _FPAD[2]), bf16),
        pltpu.VMEM((2, _B, _FPAD[3]), bf16),
        pltpu.VMEM((2, _B, _FPAD[4]), bf16),
        pltpu.VMEM((2, _B, _FPAD[5]), bf16),
        pltpu.SemaphoreType.DMA((2, _NDATA)),
        pltpu.VMEM((_N * _B, 512), jnp.float32),
    ]

    cost, card = pl.pallas_call(
        _tree_body,
        grid=(_N,),
        in_specs=in_specs,
        out_specs=out_specs,
        out_shape=out_shape,
        scratch_shapes=scratch_shapes,
        compiler_params=pltpu.CompilerParams(
            dimension_semantics=("arbitrary",),
        ),
    )(*data, *weights)
    return (cost, card)
